# baseline (speedup 1.0000x reference)
import math

import jax
import jax.numpy as jnp
from jax import lax
from jax.experimental import pallas as pl
from jax.experimental.pallas import tpu as pltpu

_NUM_EXPERTS = 64
_ROUTER_TEMP = 1.5
_LOAD_FACTOR = 0.02
_CAP_FACTOR_EVAL = 1.5

_KEY_BASE = 0x3C7F0000
_KEY_STRIDE = 0x03010001

_NROW = 32
_TB = 128


def _router_body(ks_ref, ksT_ref, ethr_ref, e_ref, val_ref,
                 comb_ref, disp_ref, ft_ref):
    step = pl.program_id(0)
    e_dim, ccap, tb = comb_ref.shape

    @pl.when(step == 0)
    def _compute_ranks():
        lane_i = lax.broadcasted_iota(jnp.int32, (1, 128), 1)
        sub_j = lax.broadcasted_iota(jnp.int32, (128, 1), 0)
        for r in range(_NROW):
            ki = ks_ref[r, :].reshape(1, 128)
            ethr_i = ethr_ref[r, :].reshape(1, 128)
            i_glob = r * 128 + lane_i
            acc = jnp.zeros((128, 128), jnp.int32)
            for jc in range(_NROW):
                kj = ksT_ref[:, jc].reshape(128, 1)
                j_glob = jc * 128 + sub_j
                gt = kj > ki
                eq = (kj == ki) & (j_glob < i_glob)
                ge = kj >= ethr_i
                acc = acc + (gt | eq).astype(jnp.int32) - ge.astype(jnp.int32)
            rank = jnp.sum(acc, axis=0, keepdims=True)
            e_row = e_ref[r, :].reshape(1, 128)
            ft = jnp.where(rank < ccap, e_row * ccap + rank, jnp.int32(-1))
            ft_ref[r, :] = ft.reshape(128)

    ft_row = ft_ref[step, :].reshape(1, 1, tb)
    val_row = val_ref[step, :].reshape(1, 1, tb)
    flat = (
        lax.broadcasted_iota(jnp.int32, (e_dim, ccap, 1), 0) * ccap
        + lax.broadcasted_iota(jnp.int32, (e_dim, ccap, 1), 1)
    )
    pred = flat == ft_row
    comb_ref[...] = jnp.where(pred, val_row, jnp.zeros((), jnp.float32))
    disp_ref[...] = pred.astype(jnp.int8)


def _pool_body(x_ref, out_ref):
    acc = x_ref[0, 0]
    for p in range(1, 49):
        acc = acc + x_ref[p // 7, p % 7]
    out_ref[...] = acc * jnp.float32(1.0 / 49.0)


def _pool(X):
    N = X.shape[0]
    C = X.shape[1]
    Xp = jnp.transpose(X, (2, 3, 1, 0))
    TB = 256
    pooled_t = pl.pallas_call(
        _pool_body,
        grid=(N // TB,),
        in_specs=[pl.BlockSpec((7, 7, C, TB), lambda i: (0, 0, 0, i))],
        out_specs=pl.BlockSpec((C, TB), lambda i: (0, i)),
        out_shape=jax.ShapeDtypeStruct((C, N), jnp.float32),
    )(Xp)
    return pooled_t.T


def kernel(X, W_gate):
    N = X.shape[0]
    E = _NUM_EXPERTS
    Ccap = max(1, math.ceil(_CAP_FACTOR_EVAL * N / E))

    pooled = _pool(X)
    logits = pooled @ W_gate
    z_loss = jnp.mean(jax.scipy.special.logsumexp(logits, axis=-1))
    probs = jax.nn.softmax(logits.astype(jnp.float32) / _ROUTER_TEMP, axis=1)
    expert_idx = jnp.argmax(probs, axis=1)
    expert_prob = jnp.take_along_axis(probs, expert_idx[:, None], axis=1)[:, 0]
    expert_mask = jax.nn.one_hot(expert_idx, E, dtype=probs.dtype)
    f_load = jnp.mean(expert_mask, axis=0)
    p_mean = jnp.mean(probs, axis=0)
    aux_loss = jnp.sum(f_load * p_mean) * E * _LOAD_FACTOR

    e32 = expert_idx.astype(jnp.int32)
    m = lax.bitcast_convert_type(expert_prob, jnp.int32)
    ku = (e32.astype(jnp.uint32) * jnp.uint32(_KEY_STRIDE)
          + (m - _KEY_BASE).astype(jnp.uint32))
    ks = lax.bitcast_convert_type(ku ^ jnp.uint32(0x80000000), jnp.int32)
    ethr_u = (e32 + 1).astype(jnp.uint32) * jnp.uint32(_KEY_STRIDE)
    ethr = lax.bitcast_convert_type(ethr_u ^ jnp.uint32(0x80000000), jnp.int32)

    ks2d = ks.reshape(_NROW, 128)
    comb_t, disp_t = pl.pallas_call(
        _router_body,
        grid=(N // _TB,),
        in_specs=[
            pl.BlockSpec((_NROW, 128), lambda i: (0, 0)),
            pl.BlockSpec((128, _NROW), lambda i: (0, 0)),
            pl.BlockSpec((_NROW, 128), lambda i: (0, 0)),
            pl.BlockSpec((_NROW, 128), lambda i: (0, 0)),
            pl.BlockSpec((_NROW, 128), lambda i: (0, 0)),
        ],
        out_specs=[
            pl.BlockSpec((E, Ccap, _TB), lambda i: (0, 0, i)),
            pl.BlockSpec((E, Ccap, _TB), lambda i: (0, 0, i)),
        ],
        out_shape=[
            jax.ShapeDtypeStruct((E, Ccap, N), jnp.float32),
            jax.ShapeDtypeStruct((E, Ccap, N), jnp.int8),
        ],
        scratch_shapes=[pltpu.VMEM((_NROW, 128), jnp.int32)],
    )(ks2d, ks2d.T, ethr.reshape(_NROW, 128), e32.reshape(_NROW, 128),
      expert_prob.reshape(_NROW, 128))

    comb = jnp.transpose(comb_t, (2, 0, 1))
    disp = jnp.transpose(disp_t, (2, 0, 1)).astype(jnp.bool_)
    return (disp, comb, z_loss, aux_loss)


# speedup vs baseline: 1.0780x; 1.0780x over previous
import math

import jax
import jax.numpy as jnp
from jax import lax
from jax.experimental import pallas as pl
from jax.experimental.pallas import tpu as pltpu

_NUM_EXPERTS = 64
_ROUTER_TEMP = 1.5
_LOAD_FACTOR = 0.02
_CAP_FACTOR_EVAL = 1.5

_KEY_BASE = 0x3C7F0000
_KEY_STRIDE = 0x03010001

_NROW = 32
_TB = 128


def _router_body(ks_ref, ksT_ref, ethr_ref, e_ref, val_ref,
                 comb_ref, disp_ref):
    step = pl.program_id(0)
    e_dim, ccap, tb = comb_ref.shape

    lane_i = lax.broadcasted_iota(jnp.int32, (1, 128), 1)
    sub_j = lax.broadcasted_iota(jnp.int32, (128, 1), 0)
    ki = ks_ref[step, :].reshape(1, 128)
    ethr_i = ethr_ref[step, :].reshape(1, 128)
    i_glob = step * 128 + lane_i
    acc = jnp.zeros((128, 128), jnp.int32)
    for jc in range(_NROW):
        kj = ksT_ref[:, jc].reshape(128, 1)
        j_glob = jc * 128 + sub_j
        gt = kj > ki
        eq = (kj == ki) & (j_glob < i_glob)
        ge = kj >= ethr_i
        acc = acc + (gt | eq).astype(jnp.int32) - ge.astype(jnp.int32)
    rank = jnp.sum(acc, axis=0, keepdims=True)
    e_row = e_ref[step, :].reshape(1, 128)
    ft_row = jnp.where(rank < ccap, e_row * ccap + rank,
                       jnp.int32(-1)).reshape(1, 1, tb)
    val_row = val_ref[step, :].reshape(1, 1, tb)
    flat = (
        lax.broadcasted_iota(jnp.int32, (e_dim, ccap, 1), 0) * ccap
        + lax.broadcasted_iota(jnp.int32, (e_dim, ccap, 1), 1)
    )
    pred = flat == ft_row
    comb_ref[...] = jnp.where(pred, val_row, jnp.zeros((), jnp.float32))
    disp_ref[...] = pred.astype(jnp.int8)


def kernel(X, W_gate):
    N = X.shape[0]
    E = _NUM_EXPERTS
    Ccap = max(1, math.ceil(_CAP_FACTOR_EVAL * N / E))

    pooled = jnp.mean(X, axis=(2, 3))
    logits = pooled @ W_gate
    z_loss = jnp.mean(jax.scipy.special.logsumexp(logits, axis=-1))
    probs = jax.nn.softmax(logits.astype(jnp.float32) / _ROUTER_TEMP, axis=1)
    expert_idx = jnp.argmax(probs, axis=1)
    expert_prob = jnp.take_along_axis(probs, expert_idx[:, None], axis=1)[:, 0]
    expert_mask = jax.nn.one_hot(expert_idx, E, dtype=probs.dtype)
    f_load = jnp.mean(expert_mask, axis=0)
    p_mean = jnp.mean(probs, axis=0)
    aux_loss = jnp.sum(f_load * p_mean) * E * _LOAD_FACTOR

    e32 = expert_idx.astype(jnp.int32)
    m = lax.bitcast_convert_type(expert_prob, jnp.int32)
    ku = (e32.astype(jnp.uint32) * jnp.uint32(_KEY_STRIDE)
          + (m - _KEY_BASE).astype(jnp.uint32))
    ks = lax.bitcast_convert_type(ku ^ jnp.uint32(0x80000000), jnp.int32)
    ethr_u = (e32 + 1).astype(jnp.uint32) * jnp.uint32(_KEY_STRIDE)
    ethr = lax.bitcast_convert_type(ethr_u ^ jnp.uint32(0x80000000), jnp.int32)

    ks2d = ks.reshape(_NROW, 128)
    comb_t, disp_t = pl.pallas_call(
        _router_body,
        grid=(N // _TB,),
        in_specs=[
            pl.BlockSpec((_NROW, 128), lambda i: (0, 0)),
            pl.BlockSpec((128, _NROW), lambda i: (0, 0)),
            pl.BlockSpec((_NROW, 128), lambda i: (0, 0)),
            pl.BlockSpec((_NROW, 128), lambda i: (0, 0)),
            pl.BlockSpec((_NROW, 128), lambda i: (0, 0)),
        ],
        out_specs=[
            pl.BlockSpec((E, Ccap, _TB), lambda i: (0, 0, i)),
            pl.BlockSpec((E, Ccap, _TB), lambda i: (0, 0, i)),
        ],
        out_shape=[
            jax.ShapeDtypeStruct((E, Ccap, N), jnp.float32),
            jax.ShapeDtypeStruct((E, Ccap, N), jnp.int8),
        ],
    )(ks2d, ks2d.T, ethr.reshape(_NROW, 128), e32.reshape(_NROW, 128),
      expert_prob.reshape(_NROW, 128))

    comb = jnp.transpose(comb_t, (2, 0, 1))
    disp = jnp.transpose(disp_t, (2, 0, 1)).astype(jnp.bool_)
    return (disp, comb, z_loss, aux_loss)


# speedup vs baseline: 1.1582x; 1.0744x over previous
import math

import jax
import jax.numpy as jnp
from jax import lax
from jax.experimental import pallas as pl
from jax.experimental.pallas import tpu as pltpu

_NUM_EXPERTS = 64
_ROUTER_TEMP = 1.5
_LOAD_FACTOR = 0.02
_CAP_FACTOR_EVAL = 1.5

_KEY_BASE = 0x3C7F0000
_KEY_STRIDE = 0x03010001

_NROW = 32
_TB = 128


def _router_body(ks_ref, ksT_ref, ethr_ref, e_ref, val_ref,
                 comb_ref, ft_ref):
    step = pl.program_id(0)
    e_dim, ccap, tb = comb_ref.shape

    lane_i = lax.broadcasted_iota(jnp.int32, (1, 128), 1)
    sub_j = lax.broadcasted_iota(jnp.int32, (128, 1), 0)
    ki = ks_ref[step, :].reshape(1, 128)
    ethr_i = ethr_ref[step, :].reshape(1, 128)
    i_glob = step * 128 + lane_i
    acc = jnp.zeros((128, 128), jnp.int32)
    for jc in range(_NROW):
        kj = ksT_ref[:, jc].reshape(128, 1)
        j_glob = jc * 128 + sub_j
        gt = kj > ki
        eq = (kj == ki) & (j_glob < i_glob)
        ge = kj >= ethr_i
        acc = acc + (gt | eq).astype(jnp.int32) - ge.astype(jnp.int32)
    rank = jnp.sum(acc, axis=0, keepdims=True)
    e_row = e_ref[step, :].reshape(1, 128)
    ft_row = jnp.where(rank < ccap, e_row * ccap + rank,
                       jnp.int32(-1)).reshape(1, 1, tb)
    ft_ref[...] = ft_row
    val_row = val_ref[step, :].reshape(1, 1, tb)
    flat = (
        lax.broadcasted_iota(jnp.int32, (e_dim, ccap, 1), 0) * ccap
        + lax.broadcasted_iota(jnp.int32, (e_dim, ccap, 1), 1)
    )
    pred = flat == ft_row
    comb_ref[...] = jnp.where(pred, val_row, jnp.zeros((), jnp.float32))


def kernel(X, W_gate):
    N = X.shape[0]
    E = _NUM_EXPERTS
    Ccap = max(1, math.ceil(_CAP_FACTOR_EVAL * N / E))

    pooled = jnp.mean(X, axis=(2, 3))
    logits = pooled @ W_gate
    z_loss = jnp.mean(jax.scipy.special.logsumexp(logits, axis=-1))
    probs = jax.nn.softmax(logits.astype(jnp.float32) / _ROUTER_TEMP, axis=1)
    expert_idx = jnp.argmax(probs, axis=1)
    expert_prob = jnp.take_along_axis(probs, expert_idx[:, None], axis=1)[:, 0]
    expert_mask = jax.nn.one_hot(expert_idx, E, dtype=probs.dtype)
    f_load = jnp.mean(expert_mask, axis=0)
    p_mean = jnp.mean(probs, axis=0)
    aux_loss = jnp.sum(f_load * p_mean) * E * _LOAD_FACTOR

    e32 = expert_idx.astype(jnp.int32)
    m = lax.bitcast_convert_type(expert_prob, jnp.int32)
    ku = (e32.astype(jnp.uint32) * jnp.uint32(_KEY_STRIDE)
          + (m - _KEY_BASE).astype(jnp.uint32))
    ks = lax.bitcast_convert_type(ku ^ jnp.uint32(0x80000000), jnp.int32)
    ethr_u = (e32 + 1).astype(jnp.uint32) * jnp.uint32(_KEY_STRIDE)
    ethr = lax.bitcast_convert_type(ethr_u ^ jnp.uint32(0x80000000), jnp.int32)

    ks2d = ks.reshape(_NROW, 128)
    comb_t, ft_out = pl.pallas_call(
        _router_body,
        grid=(N // _TB,),
        in_specs=[
            pl.BlockSpec((_NROW, 128), lambda i: (0, 0)),
            pl.BlockSpec((128, _NROW), lambda i: (0, 0)),
            pl.BlockSpec((_NROW, 128), lambda i: (0, 0)),
            pl.BlockSpec((_NROW, 128), lambda i: (0, 0)),
            pl.BlockSpec((_NROW, 128), lambda i: (0, 0)),
        ],
        out_specs=[
            pl.BlockSpec((E, Ccap, _TB), lambda i: (0, 0, i)),
            pl.BlockSpec((1, 1, _TB), lambda i: (i, 0, 0)),
        ],
        out_shape=[
            jax.ShapeDtypeStruct((E, Ccap, N), jnp.float32),
            jax.ShapeDtypeStruct((_NROW, 1, _TB), jnp.int32),
        ],
    )(ks2d, ks2d.T, ethr.reshape(_NROW, 128), e32.reshape(_NROW, 128),
      expert_prob.reshape(_NROW, 128))

    comb = jnp.transpose(comb_t, (2, 0, 1))
    flat_ec = (jnp.arange(E * Ccap, dtype=jnp.int32)
               .reshape(E, Ccap, 1))
    disp_t = flat_ec == ft_out.reshape(1, 1, N)
    disp = jnp.transpose(disp_t, (2, 0, 1))
    return (disp, comb, z_loss, aux_loss)
